# Initial kernel scaffold; baseline (speedup 1.0000x reference)
import jax
import jax.numpy as jnp
from jax.experimental import pallas as pl


def kernel(vmodel, wavelet):
    raise NotImplementedError("write your pallas kernel here")


# speedup vs baseline: 10.6236x; 10.6236x over previous
import dataclasses
import functools

import jax
import jax.numpy as jnp
from jax import lax
from jax.experimental import pallas as pl
from jax.experimental.pallas import tpu as pltpu
from jax.experimental.pallas import tpu_sc as plsc

DZ = 2.0
DTNEW = 0.001
NTNEW = 2000
NWAV = 101
ND = 2000
HALO = NWAV - 1
PW = 2304
DUMP = 2300
TT = 512
KW = TT + 128
NTILES = 4
LANES = 16


def _sc_scatter(refl, sidx, batch):
    mesh = plsc.VectorSubcoreMesh(core_axis_name="c", subcore_axis_name="s")
    cp = pltpu.CompilerParams()
    if "needs_layout_passes" in pltpu.CompilerParams.__dataclass_fields__:
        cp = dataclasses.replace(cp, needs_layout_passes=False)

    @functools.partial(
        pl.kernel,
        out_type=jax.ShapeDtypeStruct((batch, PW), jnp.float32),
        mesh=mesh,
        compiler_params=cp,
    )
    def scatter_kernel(refl_hbm, sidx_hbm, p_hbm):
        def body(refl_v, sidx_v, p_v):
            row = p_v.at[0]
            zero = jnp.zeros((LANES,), jnp.float32)

            @pl.loop(0, PW, step=LANES)
            def _zero(c):
                row.at[pl.ds(c, LANES)][...] = zero

            @pl.loop(0, ND, step=LANES)
            def _scatter(i):
                vals = refl_v[0, pl.ds(i, LANES)]
                idxs = sidx_v[0, pl.ds(i, LANES)]
                mask = idxs < DUMP
                plsc.store_scatter(row, [idxs], vals, mask=mask)

        pltpu.emit_pipeline(
            body,
            grid=(batch,),
            in_specs=[
                pl.BlockSpec((1, ND), lambda i: (i, 0)),
                pl.BlockSpec((1, ND), lambda i: (i, 0)),
            ],
            out_specs=[pl.BlockSpec((1, PW), lambda i: (i, 0))],
            core_axis_name=("c", "s"),
            dimension_semantics=(pltpu.PARALLEL,),
        )(refl_hbm, sidx_hbm, p_hbm)

    return scatter_kernel(refl, sidx)


def _conv_body(p_ref, w_ref, o_ref):
    w = w_ref[...]
    for j in range(NTILES):
        o_ref[:, j * TT:(j + 1) * TT] = jnp.dot(
            p_ref[:, j * TT:j * TT + KW], w,
            preferred_element_type=jnp.float32)


def _tc_conv(p, wband, batch):
    bm = 256
    return pl.pallas_call(
        _conv_body,
        grid=(batch // bm,),
        in_specs=[
            pl.BlockSpec((bm, PW), lambda i: (i, 0)),
            pl.BlockSpec((KW, TT), lambda i: (0, 0)),
        ],
        out_specs=pl.BlockSpec((bm, NTILES * TT), lambda i: (i, 0)),
        out_shape=jax.ShapeDtypeStruct((batch, NTILES * TT), jnp.float32),
    )(p, wband)


def kernel(vmodel, wavelet):
    batch = vmodel.shape[0]
    v0 = vmodel[:, :-1]
    v1 = vmodel[:, 1:]
    refl = (v1 - v0) / (v1 + v0)
    time_old = jnp.cumsum(DZ / v0, axis=1)
    idx = jnp.round(time_old / DTNEW).astype(jnp.int32)
    valid = (idx >= 0) & (idx < NTNEW)
    nxt = jnp.concatenate(
        [idx[:, 1:], jnp.full((batch, 1), -1, jnp.int32)], axis=1)
    last_in_bin = idx != nxt
    sidx = jnp.where(valid & last_in_bin, idx + HALO, DUMP)

    p = _sc_scatter(refl, sidx, batch)

    k = lax.broadcasted_iota(jnp.int32, (KW, TT), 0)
    u = lax.broadcasted_iota(jnp.int32, (KW, TT), 1)
    j = u - k + HALO
    band = (j >= 0) & (j <= HALO)
    wband = jnp.where(band, jnp.take(wavelet, jnp.clip(j, 0, HALO)), 0.0)
    wband = wband.astype(jnp.float32)

    out = _tc_conv(p, wband, batch)
    return out[:, :NTNEW]


# speedup vs baseline: 12.0035x; 1.1299x over previous
import dataclasses
import functools

import jax
import jax.numpy as jnp
from jax import lax
from jax.experimental import pallas as pl
from jax.experimental.pallas import tpu as pltpu
from jax.experimental.pallas import tpu_sc as plsc

DZ = 2.0
DTNEW = 0.001
NTNEW = 2000
NWAV = 101
ND = 2000
NV = 2001
HALO = NWAV - 1
PW = 2304
TT = 512
KW = TT + 128
NTILES = 4
LANES = 16
NWORKERS = 32
CSBLK = 128
MAGIC = float(1.5 * 2**23)


def _sc_fused(vmodel, batch):
    mesh = plsc.VectorSubcoreMesh(core_axis_name="c", subcore_axis_name="s")
    cp = pltpu.CompilerParams()
    if "needs_layout_passes" in pltpu.CompilerParams.__dataclass_fields__:
        cp = dataclasses.replace(cp, needs_layout_passes=False)
    rows_per_worker = batch // NWORKERS
    groups = rows_per_worker // LANES
    nblocks = [CSBLK] * (ND // CSBLK) + ([ND % CSBLK] if ND % CSBLK else [])

    @functools.partial(
        pl.kernel,
        out_type=jax.ShapeDtypeStruct((batch, PW), jnp.float32),
        mesh=mesh,
        compiler_params=cp,
        scratch_types=[
            pltpu.VMEM((LANES, NV), jnp.float32),
            pltpu.VMEM((LANES, PW), jnp.float32),
        ],
    )
    def k(v_hbm, p_hbm, vbuf, ptime):
        wid = lax.axis_index("s") * 2 + lax.axis_index("c")
        lanes = lax.broadcasted_iota(jnp.int32, (LANES,), 0)
        zeros = jnp.zeros((LANES,), jnp.float32)

        @pl.loop(0, groups)
        def _group(g):
            row0 = (wid * groups + g) * LANES
            pltpu.sync_copy(v_hbm.at[pl.ds(row0, LANES), :], vbuf)

            for r in range(LANES):
                @pl.loop(0, PW, step=LANES)
                def _zero(cc, r=r):
                    ptime.at[r, pl.ds(cc, LANES)][...] = zeros

            vcur0 = plsc.load_gather(vbuf, [lanes, jnp.zeros((LANES,), jnp.int32)])
            carry = (
                vcur0,
                jnp.ones((LANES,), jnp.int32),
                zeros,
                zeros,
                jnp.full((LANES,), 2**20, jnp.int32),
                zeros,
            )

            def step(j, carry):
                vcur, colv, w, c, prev_idx, prev_refl = carry
                vnext = plsc.load_gather(vbuf, [lanes, colv])
                w = w + DZ / vcur
                x = (c + w) / DTNEW
                idx = ((x + MAGIC) - MAGIC).astype(jnp.int32)
                refl = (vnext - vcur) / (vnext + vcur)
                m = (prev_idx != idx) & (prev_idx < NTNEW) & (prev_idx >= 0)
                plsc.store_scatter(
                    ptime, [lanes, prev_idx + HALO], prev_refl, mask=m)
                return (vnext, colv + 1, w, c, idx, refl)

            for nb in nblocks:
                carry = lax.fori_loop(0, nb, step, carry, unroll=False)
                vcur, colv, w, c, prev_idx, prev_refl = carry
                carry = (vcur, colv, zeros, c + w, prev_idx, prev_refl)

            _, _, _, _, prev_idx, prev_refl = carry
            m = (prev_idx < NTNEW) & (prev_idx >= 0)
            plsc.store_scatter(
                ptime, [lanes, prev_idx + HALO], prev_refl, mask=m)

            pltpu.sync_copy(ptime, p_hbm.at[pl.ds(row0, LANES), :])

    return k(vmodel)


def _conv_body(p_ref, w_ref, o_ref):
    w = w_ref[...]
    for j in range(NTILES):
        o_ref[:, j * TT:(j + 1) * TT] = jnp.dot(
            p_ref[:, j * TT:j * TT + KW], w,
            preferred_element_type=jnp.float32)


def _tc_conv(p, wband, batch):
    bm = 256
    return pl.pallas_call(
        _conv_body,
        grid=(batch // bm,),
        in_specs=[
            pl.BlockSpec((bm, PW), lambda i: (i, 0)),
            pl.BlockSpec((KW, TT), lambda i: (0, 0)),
        ],
        out_specs=pl.BlockSpec((bm, NTILES * TT), lambda i: (i, 0)),
        out_shape=jax.ShapeDtypeStruct((batch, NTILES * TT), jnp.float32),
    )(p, wband)


def kernel(vmodel, wavelet):
    batch = vmodel.shape[0]
    p = _sc_fused(vmodel, batch)

    k = lax.broadcasted_iota(jnp.int32, (KW, TT), 0)
    u = lax.broadcasted_iota(jnp.int32, (KW, TT), 1)
    j = u - k + HALO
    band = (j >= 0) & (j <= HALO)
    wband = jnp.where(band, jnp.take(wavelet, jnp.clip(j, 0, HALO)), 0.0)
    wband = wband.astype(jnp.float32)

    out = _tc_conv(p, wband, batch)
    return out[:, :NTNEW]
